# baseline (speedup 1.0000x reference)
import functools

import jax
import jax.numpy as jnp
from jax.experimental import pallas as pl
from jax.experimental.pallas import tpu as pltpu

_N = 10000
_D = 128


def _proj_body(feat_ref, w1_ref, g_ref):
    f = feat_ref[...].astype(jnp.bfloat16)
    w = w1_ref[...].astype(jnp.bfloat16)
    g = jax.lax.dot_general(
        f, w, (((1,), (0,)), ((), ())), preferred_element_type=jnp.float32
    )
    g_ref[...] = g.astype(jnp.bfloat16)


def _pass1_body(adj_ref, g_ref, b1_ref, w2_ref, h2_ref):
    a = adj_ref[...].astype(jnp.bfloat16)
    acc = jax.lax.dot_general(
        a, g_ref[...], (((1,), (0,)), ((), ())), preferred_element_type=jnp.float32
    )
    h = jnp.tanh(acc + b1_ref[...])
    h2 = jax.lax.dot_general(
        h.astype(jnp.bfloat16),
        w2_ref[...].astype(jnp.bfloat16),
        (((1,), (0,)), ((), ())),
        preferred_element_type=jnp.float32,
    )
    h2_ref[...] = h2.astype(jnp.bfloat16)


def _pass2_body(adj_ref, h2_ref, out_ref):
    a = adj_ref[...].astype(jnp.bfloat16)
    out_ref[...] = jax.lax.dot_general(
        a, h2_ref[...], (((1,), (0,)), ((), ())), preferred_element_type=jnp.float32
    )


@functools.partial(jax.jit, static_argnames=("bm",))
def _run(adj, feat, W1, b1, W2, bm=512):
    n, d = _N, _D
    num_blocks = pl.cdiv(n, bm)

    g = pl.pallas_call(
        _proj_body,
        grid=(10,),
        in_specs=[
            pl.BlockSpec((1000, d), lambda i: (i, 0)),
            pl.BlockSpec((d, d), lambda i: (0, 0)),
        ],
        out_specs=pl.BlockSpec((1000, d), lambda i: (i, 0)),
        out_shape=jax.ShapeDtypeStruct((n, d), jnp.bfloat16),
        compiler_params=pltpu.CompilerParams(
            dimension_semantics=("parallel",),
        ),
    )(feat, W1)

    b1_2d = b1.reshape(1, d)

    h2 = pl.pallas_call(
        _pass1_body,
        grid=(num_blocks,),
        in_specs=[
            pl.BlockSpec((bm, n), lambda i: (i, 0)),
            pl.BlockSpec((n, d), lambda i: (0, 0)),
            pl.BlockSpec((1, d), lambda i: (0, 0)),
            pl.BlockSpec((d, d), lambda i: (0, 0)),
        ],
        out_specs=pl.BlockSpec((bm, d), lambda i: (i, 0)),
        out_shape=jax.ShapeDtypeStruct((n, d), jnp.bfloat16),
        compiler_params=pltpu.CompilerParams(
            dimension_semantics=("parallel",),
        ),
    )(adj, g, b1_2d, W2)

    out = pl.pallas_call(
        _pass2_body,
        grid=(num_blocks,),
        in_specs=[
            pl.BlockSpec((bm, n), lambda i: (i, 0)),
            pl.BlockSpec((n, d), lambda i: (0, 0)),
        ],
        out_specs=pl.BlockSpec((bm, d), lambda i: (i, 0)),
        out_shape=jax.ShapeDtypeStruct((n, d), jnp.float32),
        compiler_params=pltpu.CompilerParams(
            dimension_semantics=("parallel",),
        ),
    )(adj, h2)
    return out


def kernel(adj, feat, W1, b1, W2):
    return _run(adj, feat, W1, b1, W2)


# speedup vs baseline: 1.0250x; 1.0250x over previous
import functools

import jax
import jax.numpy as jnp
from jax.experimental import pallas as pl
from jax.experimental.pallas import tpu as pltpu

_N = 10000
_D = 128
_BM = 400


def _proj_body(feat_ref, w1_ref, g_ref):
    f = feat_ref[...].astype(jnp.bfloat16)
    w = w1_ref[...].astype(jnp.bfloat16)
    g = jax.lax.dot_general(
        f, w, (((1,), (0,)), ((), ())), preferred_element_type=jnp.float32
    )
    g_ref[...] = g.astype(jnp.bfloat16)


def _fused_body(adj_ref, g_ref, b1_ref, w2_ref, out_ref, h2_ref):
    p = pl.program_id(0)
    i = pl.program_id(1)
    a = adj_ref[...].astype(jnp.bfloat16)

    @pl.when(p == 0)
    def _phase0():
        acc = jax.lax.dot_general(
            a, g_ref[...], (((1,), (0,)), ((), ())),
            preferred_element_type=jnp.float32,
        )
        h = jnp.tanh(acc + b1_ref[...])
        h2 = jax.lax.dot_general(
            h.astype(jnp.bfloat16),
            w2_ref[...].astype(jnp.bfloat16),
            (((1,), (0,)), ((), ())),
            preferred_element_type=jnp.float32,
        )
        h2_ref[pl.ds(i * _BM, _BM), :] = h2.astype(jnp.bfloat16)

    @pl.when(p == 1)
    def _phase1():
        out_ref[...] = jax.lax.dot_general(
            a, h2_ref[...], (((1,), (0,)), ((), ())),
            preferred_element_type=jnp.float32,
        )


@jax.jit
def _run(adj, feat, W1, b1, W2):
    n, d, bm = _N, _D, _BM
    num_blocks = n // bm

    g = pl.pallas_call(
        _proj_body,
        grid=(10,),
        in_specs=[
            pl.BlockSpec((n // 10, d), lambda i: (i, 0)),
            pl.BlockSpec((d, d), lambda i: (0, 0)),
        ],
        out_specs=pl.BlockSpec((n // 10, d), lambda i: (i, 0)),
        out_shape=jax.ShapeDtypeStruct((n, d), jnp.bfloat16),
    )(feat, W1)

    b1_2d = b1.reshape(1, d)

    out = pl.pallas_call(
        _fused_body,
        grid=(2, num_blocks),
        in_specs=[
            pl.BlockSpec((bm, n), lambda p, i: (i, 0)),
            pl.BlockSpec((n, d), lambda p, i: (0, 0)),
            pl.BlockSpec((1, d), lambda p, i: (0, 0)),
            pl.BlockSpec((d, d), lambda p, i: (0, 0)),
        ],
        out_specs=pl.BlockSpec((bm, d), lambda p, i: (i, 0)),
        out_shape=jax.ShapeDtypeStruct((n, d), jnp.float32),
        scratch_shapes=[pltpu.VMEM((n, d), jnp.bfloat16)],
    )(adj, g, b1_2d, W2)
    return out


def kernel(adj, feat, W1, b1, W2):
    return _run(adj, feat, W1, b1, W2)
